# Initial kernel scaffold; baseline (speedup 1.0000x reference)
import jax
import jax.numpy as jnp
from jax.experimental import pallas as pl


def kernel(x, edge_index, W1, a_src, a_dst, b1, gamma, beta, W2, b2):
    raise NotImplementedError("write your pallas kernel here")


# speedup vs baseline: 40.0261x; 40.0261x over previous
import functools

import jax
import jax.numpy as jnp
from jax import lax
from jax.experimental import pallas as pl
from jax.experimental.pallas import tpu as pltpu
from jax.experimental.pallas import tpu_sc as plsc

N = 10000
E = 320000
D = 128
H = 64

NC = 2
NS = 16
NW = NC * NS
C = 80
NCH = E // (NW * C)
NP = 10240
RPT = NP // NS
ZB = RPT // C
BR = 512
L = 16

_mesh = plsc.VectorSubcoreMesh(
    core_axis_name="c", subcore_axis_name="s", num_cores=NC, num_subcores=NS)


def _tc_pre_body(x_ref, w1_ref, asrc_ref, adst_ref,
                 h_ref, as_ref, ad_ref, pl_ref):
  h = jnp.dot(x_ref[...], w1_ref[...], preferred_element_type=jnp.float32)
  h_ref[...] = h
  s = jnp.sum(h * asrc_ref[...], axis=1, keepdims=True)
  d = jnp.sum(h * adst_ref[...], axis=1, keepdims=True)
  as_ref[...] = s
  ad_ref[...] = d
  z = s + d
  z = jnp.maximum(z, 0.2 * z)
  pl_ref[...] = jnp.exp(z)


def _tc_pre(xp, W1, a_src, a_dst):
  grid = (NP // BR,)
  return pl.pallas_call(
      _tc_pre_body,
      grid=grid,
      in_specs=[
          pl.BlockSpec((BR, D), lambda i: (i, 0)),
          pl.BlockSpec((D, H), lambda i: (0, 0)),
          pl.BlockSpec((1, H), lambda i: (0, 0)),
          pl.BlockSpec((1, H), lambda i: (0, 0)),
      ],
      out_specs=[
          pl.BlockSpec((BR, H), lambda i: (i, 0)),
          pl.BlockSpec((BR, 1), lambda i: (i, 0)),
          pl.BlockSpec((BR, 1), lambda i: (i, 0)),
          pl.BlockSpec((BR, 1), lambda i: (i, 0)),
      ],
      out_shape=[
          jax.ShapeDtypeStruct((NP, H), jnp.float32),
          jax.ShapeDtypeStruct((NP, 1), jnp.float32),
          jax.ShapeDtypeStruct((NP, 1), jnp.float32),
          jax.ShapeDtypeStruct((NP, 1), jnp.float32),
      ],
  )(xp, W1, a_src, a_dst)


@functools.partial(
    pl.kernel,
    out_type=[
        jax.ShapeDtypeStruct((NC, NP, H), jnp.float32),
        jax.ShapeDtypeStruct((NP,), jnp.float32),
        jax.ShapeDtypeStruct((NP,), jnp.float32),
        jax.ShapeDtypeStruct((NP,), jnp.float32),
        jax.ShapeDtypeStruct((NP,), jnp.float32),
    ],
    mesh=_mesh,
    scratch_types=[
        pltpu.VMEM((NCH, C), jnp.int32),
        pltpu.VMEM((NCH, C), jnp.int32),
        pltpu.VMEM((C,), jnp.float32),
        pltpu.VMEM((C,), jnp.float32),
        pltpu.VMEM((C,), jnp.float32),
        pltpu.VMEM((C,), jnp.float32),
        pltpu.VMEM((C, H), jnp.float32),
        pltpu.VMEM((C, H), jnp.float32),
        pltpu.VMEM((C,), jnp.float32),
        pltpu.VMEM_SHARED((NP, H), jnp.float32),
        pltpu.VMEM_SHARED((NP,), jnp.float32),
        pltpu.VMEM_SHARED((NP,), jnp.float32),
        pltpu.SemaphoreType.DMA,
        pltpu.SemaphoreType.DMA,
        pltpu.SemaphoreType.DMA,
    ],
    compiler_params=pltpu.CompilerParams(use_tc_tiling_on_sc=False),
)
def _sc_gat(src_hbm, dst_hbm, as_hbm, ad_hbm, h_hbm,
            acc_out, den0_out, den1_out, deg0_out, deg1_out,
            sidx, didx, aval_s, aval_d, pbuf, ones, hrows, zbuf, zflat,
            acc_sh, denom_sh, deg_sh, sem0, sem1, sem2):
  c = lax.axis_index("c")
  s = lax.axis_index("s")
  wid = c * NS + s

  zero16 = jnp.zeros((L,), jnp.float32)
  one16 = jnp.ones((L,), jnp.float32)

  def initrow(i, _):
    for q in range(H // L):
      zbuf[i, pl.ds(q * L, L)] = zero16
    return 0
  lax.fori_loop(0, C, initrow, 0)
  for v in range(C // L):
    ones[pl.ds(v * L, L)] = one16
    zflat[pl.ds(v * L, L)] = zero16

  base = s * RPT
  for k in range(ZB):
    pltpu.sync_copy(zbuf, acc_sh.at[pl.ds(base + k * C, C)])
    pltpu.sync_copy(zflat, denom_sh.at[pl.ds(base + k * C, C)])
    pltpu.sync_copy(zflat, deg_sh.at[pl.ds(base + k * C, C)])
  plsc.subcore_barrier()

  pltpu.sync_copy(src_hbm.at[wid], sidx)
  pltpu.sync_copy(dst_hbm.at[wid], didx)

  def chunk(j, _):
    isrc = sidx.at[j]
    idst = didx.at[j]
    ga = pltpu.async_copy(as_hbm.at[isrc], aval_s, sem0)
    gb = pltpu.async_copy(ad_hbm.at[idst], aval_d, sem1)
    gh = pltpu.async_copy(h_hbm.at[isrc], hrows, sem2)
    ga.wait()
    gb.wait()
    pvs = []
    for v in range(C // L):
      e = aval_s[pl.ds(v * L, L)] + aval_d[pl.ds(v * L, L)]
      e = jnp.maximum(e, 0.2 * e)
      pv = jnp.exp(e)
      pbuf[pl.ds(v * L, L)] = pv
      pvs.append(pv)
    gh.wait()
    for v in range(C // L):
      pv = pvs[v]
      for l in range(L):
        bc = jnp.full((L,), pv[l], jnp.float32)
        i = v * L + l
        for q in range(H // L):
          hrows[i, pl.ds(q * L, L)] = hrows[i, pl.ds(q * L, L)] * bc

    pltpu.sync_copy(pbuf, denom_sh.at[idst], add=True)
    pltpu.sync_copy(ones, deg_sh.at[idst], add=True)
    pltpu.sync_copy(hrows, acc_sh.at[idst], add=True)
    return 0
  lax.fori_loop(0, NCH, chunk, 0)
  plsc.subcore_barrier()

  for k in range(ZB):
    r0 = base + k * C
    pltpu.sync_copy(acc_sh.at[pl.ds(r0, C)], zbuf)
    pltpu.sync_copy(zbuf, acc_out.at[c, pl.ds(r0, C)])

  def _drain_scalars(den_out, deg_out):
    for k in range(ZB):
      r0 = base + k * C
      pltpu.sync_copy(denom_sh.at[pl.ds(r0, C)], zflat)
      pltpu.sync_copy(zflat, den_out.at[pl.ds(r0, C)])
      pltpu.sync_copy(deg_sh.at[pl.ds(r0, C)], zflat)
      pltpu.sync_copy(zflat, deg_out.at[pl.ds(r0, C)])

  @pl.when(c == 0)
  def _():
    _drain_scalars(den0_out, deg0_out)

  @pl.when(c == 1)
  def _():
    _drain_scalars(den1_out, deg1_out)


def _tc_mid_body(acc0, acc1, den0, den1, deg0, deg1, h, ploop,
                 b1, gamma, beta, w2t, g_ref, dinv_ref):
  p = ploop[...]
  acc = acc0[...] + acc1[...] + p * h[...]
  den = den0[...] + den1[...] + p
  o = acc / (den + 1e-16) + b1[...]
  r = jnp.maximum(o, 0.0)
  mu = jnp.mean(r, axis=1, keepdims=True)
  var = jnp.mean((r - mu) ** 2, axis=1, keepdims=True)
  y = (r - mu) * lax.rsqrt(var + 1e-5) * gamma[...] + beta[...]
  h2 = jnp.sum(y * w2t[...], axis=1, keepdims=True)
  deg = deg0[...] + deg1[...] + 1.0
  dinv = lax.rsqrt(jnp.maximum(deg, 1.0))
  g_ref[...] = dinv * h2
  dinv_ref[...] = dinv


def _tc_mid(acc0, acc1, den0, den1, deg0, deg1, h, ploop, b1, gamma, beta, w2t):
  grid = (NP // BR,)
  row = lambda i: (i, 0)
  const = lambda i: (0, 0)
  return pl.pallas_call(
      _tc_mid_body,
      grid=grid,
      in_specs=[
          pl.BlockSpec((BR, H), row),
          pl.BlockSpec((BR, H), row),
          pl.BlockSpec((BR, 1), row),
          pl.BlockSpec((BR, 1), row),
          pl.BlockSpec((BR, 1), row),
          pl.BlockSpec((BR, 1), row),
          pl.BlockSpec((BR, H), row),
          pl.BlockSpec((BR, 1), row),
          pl.BlockSpec((1, H), const),
          pl.BlockSpec((1, H), const),
          pl.BlockSpec((1, H), const),
          pl.BlockSpec((1, H), const),
      ],
      out_specs=[
          pl.BlockSpec((BR, 1), row),
          pl.BlockSpec((BR, 1), row),
      ],
      out_shape=[
          jax.ShapeDtypeStruct((NP, 1), jnp.float32),
          jax.ShapeDtypeStruct((NP, 1), jnp.float32),
      ],
  )(acc0, acc1, den0, den1, deg0, deg1, h, ploop, b1, gamma, beta, w2t)


@functools.partial(
    pl.kernel,
    out_type=[
        jax.ShapeDtypeStruct((NP,), jnp.float32),
        jax.ShapeDtypeStruct((NP,), jnp.float32),
    ],
    mesh=_mesh,
    scratch_types=[
        pltpu.VMEM((NCH, C), jnp.int32),
        pltpu.VMEM((NCH, C), jnp.int32),
        pltpu.VMEM((C,), jnp.float32),
        pltpu.VMEM((C,), jnp.float32),
        pltpu.VMEM_SHARED((NP,), jnp.float32),
        pltpu.SemaphoreType.DMA,
    ],
    compiler_params=pltpu.CompilerParams(use_tc_tiling_on_sc=False),
)
def _sc_gcn(src_hbm, dst_hbm, g_hbm, acc2_0_out, acc2_1_out,
            sidx, didx, gval, zflat, acc2_sh, sem0):
  c = lax.axis_index("c")
  s = lax.axis_index("s")
  wid = c * NS + s

  zero16 = jnp.zeros((L,), jnp.float32)
  for v in range(C // L):
    zflat[pl.ds(v * L, L)] = zero16
  base = s * RPT
  for k in range(ZB):
    pltpu.sync_copy(zflat, acc2_sh.at[pl.ds(base + k * C, C)])
  plsc.subcore_barrier()

  pltpu.sync_copy(src_hbm.at[wid], sidx)
  pltpu.sync_copy(dst_hbm.at[wid], didx)

  def chunk(j, _):
    pltpu.async_copy(g_hbm.at[sidx.at[j]], gval, sem0).wait()
    pltpu.sync_copy(gval, acc2_sh.at[didx.at[j]], add=True)
    return 0
  lax.fori_loop(0, NCH, chunk, 0)
  plsc.subcore_barrier()

  def _drain(out_ref):
    for k in range(ZB):
      r0 = base + k * C
      pltpu.sync_copy(acc2_sh.at[pl.ds(r0, C)], zflat)
      pltpu.sync_copy(zflat, out_ref.at[pl.ds(r0, C)])

  @pl.when(c == 0)
  def _():
    _drain(acc2_0_out)

  @pl.when(c == 1)
  def _():
    _drain(acc2_1_out)


def _tc_fin_body(a0, a1, g, dinv, b2, out_ref):
  out_ref[...] = dinv[...] * (a0[...] + a1[...] + g[...]) + b2[...]


def _tc_fin(a0, a1, g, dinv, b2):
  grid = (NP // BR,)
  row = lambda i: (i, 0)
  return pl.pallas_call(
      _tc_fin_body,
      grid=grid,
      in_specs=[
          pl.BlockSpec((BR, 1), row),
          pl.BlockSpec((BR, 1), row),
          pl.BlockSpec((BR, 1), row),
          pl.BlockSpec((BR, 1), row),
          pl.BlockSpec((1, 1), lambda i: (0, 0)),
      ],
      out_specs=pl.BlockSpec((BR, 1), row),
      out_shape=jax.ShapeDtypeStruct((NP, 1), jnp.float32),
  )(a0, a1, g, dinv, b2)


def kernel(x, edge_index, W1, a_src, a_dst, b1, gamma, beta, W2, b2):
  xp = jnp.pad(x, ((0, NP - N), (0, 0)))
  src3 = edge_index[0].reshape(NW, NCH, C)
  dst3 = edge_index[1].reshape(NW, NCH, C)

  h, as_, ad_, ploop = _tc_pre(xp, W1, a_src, a_dst)

  acc_p, den0, den1, deg0, deg1 = _sc_gat(
      src3, dst3, as_.reshape(NP), ad_.reshape(NP), h)

  g2, dinv2 = _tc_mid(
      acc_p[0], acc_p[1],
      den0.reshape(NP, 1), den1.reshape(NP, 1),
      deg0.reshape(NP, 1), deg1.reshape(NP, 1),
      h, ploop,
      b1.reshape(1, H), gamma.reshape(1, H), beta.reshape(1, H),
      W2.reshape(1, H))

  acc2_0, acc2_1 = _sc_gcn(src3, dst3, g2.reshape(NP))

  outp = _tc_fin(acc2_0.reshape(NP, 1), acc2_1.reshape(NP, 1),
                 g2, dinv2, b2.reshape(1, 1))
  return outp[:N, 0]


# speedup vs baseline: 49.3081x; 1.2319x over previous
import functools

import jax
import jax.numpy as jnp
from jax import lax
from jax.experimental import pallas as pl
from jax.experimental.pallas import tpu as pltpu
from jax.experimental.pallas import tpu_sc as plsc

N = 10000
E = 320000
D = 128
H = 64

NC = 2
NS = 16
NW = NC * NS
C = 80
NCH = E // (NW * C)
NP = 10240
RPT = NP // NS
ZB = RPT // C
BR = 512
L = 16
NBUF = 5
NJJ = NCH // NBUF

_mesh = plsc.VectorSubcoreMesh(
    core_axis_name="c", subcore_axis_name="s", num_cores=NC, num_subcores=NS)


def _tc_pre_body(x_ref, w1_ref, asrc_ref, adst_ref,
                 h_ref, as_ref, ad_ref, pl_ref):
  h = jnp.dot(x_ref[...], w1_ref[...], preferred_element_type=jnp.float32)
  h_ref[...] = h
  s = jnp.sum(h * asrc_ref[...], axis=1, keepdims=True)
  d = jnp.sum(h * adst_ref[...], axis=1, keepdims=True)
  as_ref[...] = s
  ad_ref[...] = d
  z = s + d
  z = jnp.maximum(z, 0.2 * z)
  pl_ref[...] = jnp.exp(z)


def _tc_pre(xp, W1, a_src, a_dst):
  grid = (NP // BR,)
  return pl.pallas_call(
      _tc_pre_body,
      grid=grid,
      in_specs=[
          pl.BlockSpec((BR, D), lambda i: (i, 0)),
          pl.BlockSpec((D, H), lambda i: (0, 0)),
          pl.BlockSpec((1, H), lambda i: (0, 0)),
          pl.BlockSpec((1, H), lambda i: (0, 0)),
      ],
      out_specs=[
          pl.BlockSpec((BR, H), lambda i: (i, 0)),
          pl.BlockSpec((BR, 1), lambda i: (i, 0)),
          pl.BlockSpec((BR, 1), lambda i: (i, 0)),
          pl.BlockSpec((BR, 1), lambda i: (i, 0)),
      ],
      out_shape=[
          jax.ShapeDtypeStruct((NP, H), jnp.float32),
          jax.ShapeDtypeStruct((NP, 1), jnp.float32),
          jax.ShapeDtypeStruct((NP, 1), jnp.float32),
          jax.ShapeDtypeStruct((NP, 1), jnp.float32),
      ],
  )(xp, W1, a_src, a_dst)


@functools.partial(
    pl.kernel,
    out_type=[
        jax.ShapeDtypeStruct((NC, NP, H), jnp.float32),
        jax.ShapeDtypeStruct((NP,), jnp.float32),
        jax.ShapeDtypeStruct((NP,), jnp.float32),
        jax.ShapeDtypeStruct((NP,), jnp.float32),
        jax.ShapeDtypeStruct((NP,), jnp.float32),
    ],
    mesh=_mesh,
    scratch_types=[
        pltpu.VMEM((NCH, C), jnp.int32),
        pltpu.VMEM((NCH, C), jnp.int32),
        pltpu.VMEM((NBUF, C), jnp.float32),
        pltpu.VMEM((NBUF, C), jnp.float32),
        pltpu.VMEM((NBUF, C), jnp.float32),
        pltpu.VMEM((C,), jnp.float32),
        pltpu.VMEM((NBUF, C, H), jnp.float32),
        pltpu.VMEM((NBUF, C, H), jnp.float32),
        pltpu.VMEM((C, H), jnp.float32),
        pltpu.VMEM((C,), jnp.float32),
        pltpu.VMEM_SHARED((NP, H), jnp.float32),
        pltpu.VMEM_SHARED((NP,), jnp.float32),
        pltpu.VMEM_SHARED((NP,), jnp.float32),
    ] + [pltpu.SemaphoreType.DMA] * (2 * NBUF),
    compiler_params=pltpu.CompilerParams(use_tc_tiling_on_sc=False),
)
def _sc_gat(src_hbm, dst_hbm, as_hbm, ad_hbm, h_hbm,
            acc_out, den0_out, den1_out, deg0_out, deg1_out,
            sidx, didx, aval_s, aval_d, pbuf, ones, hrows, srows, zbuf, zflat,
            acc_sh, denom_sh, deg_sh, *sems):
  gsem = sems[:NBUF]
  ssem = sems[NBUF:]
  c = lax.axis_index("c")
  s = lax.axis_index("s")
  wid = c * NS + s

  zero16 = jnp.zeros((L,), jnp.float32)
  one16 = jnp.ones((L,), jnp.float32)

  def initrow(i, _):
    for q in range(H // L):
      zbuf[i, pl.ds(q * L, L)] = zero16
    return 0
  lax.fori_loop(0, C, initrow, 0)
  for v in range(C // L):
    ones[pl.ds(v * L, L)] = one16
    zflat[pl.ds(v * L, L)] = zero16

  base = s * RPT
  for k in range(ZB):
    pltpu.sync_copy(zbuf, acc_sh.at[pl.ds(base + k * C, C)])
    pltpu.sync_copy(zflat, denom_sh.at[pl.ds(base + k * C, C)])
    pltpu.sync_copy(zflat, deg_sh.at[pl.ds(base + k * C, C)])
  plsc.subcore_barrier()

  pltpu.sync_copy(src_hbm.at[wid], sidx)
  pltpu.sync_copy(dst_hbm.at[wid], didx)

  def fire_gather(j, b):
    pltpu.async_copy(as_hbm.at[sidx.at[j]], aval_s.at[b], gsem[b])
    pltpu.async_copy(ad_hbm.at[didx.at[j]], aval_d.at[b], gsem[b])
    pltpu.async_copy(h_hbm.at[sidx.at[j]], hrows.at[b], gsem[b])

  def wait_gather(j, b):
    pltpu.make_async_copy(as_hbm.at[sidx.at[j]], aval_s.at[b], gsem[b]).wait()
    pltpu.make_async_copy(ad_hbm.at[didx.at[j]], aval_d.at[b], gsem[b]).wait()
    pltpu.make_async_copy(h_hbm.at[sidx.at[j]], hrows.at[b], gsem[b]).wait()

  def fire_scatter(j, b):
    pltpu.async_copy(pbuf.at[b], denom_sh.at[didx.at[j]], ssem[b], add=True)
    pltpu.async_copy(ones, deg_sh.at[didx.at[j]], ssem[b], add=True)
    pltpu.async_copy(srows.at[b], acc_sh.at[didx.at[j]], ssem[b], add=True)

  def wait_scatter(j, b):
    pltpu.make_async_copy(pbuf.at[b], denom_sh.at[didx.at[j]], ssem[b]).wait()
    pltpu.make_async_copy(ones, deg_sh.at[didx.at[j]], ssem[b]).wait()
    pltpu.make_async_copy(srows.at[b], acc_sh.at[didx.at[j]], ssem[b]).wait()

  for b in range(NBUF):
    fire_gather(b, b)

  def outer(jj, _):
    for b in range(NBUF):
      j = jj * NBUF + b
      wait_gather(j, b)

      @pl.when(jj > 0)
      def _():
        wait_scatter(j - NBUF, b)

      def group(v, _):
        sv = aval_s[b, pl.ds(v * L, L)]
        dv = aval_d[b, pl.ds(v * L, L)]
        e = sv + dv
        e = jnp.maximum(e, 0.2 * e)
        pv = jnp.exp(e)
        pbuf[b, pl.ds(v * L, L)] = pv
        for l in range(L):
          bc = jnp.full((L,), pv[l], jnp.float32)
          r = v * L + l
          for q in range(H // L):
            srows[b, r, pl.ds(q * L, L)] = hrows[b, r, pl.ds(q * L, L)] * bc
        return 0
      lax.fori_loop(0, C // L, group, 0)

      @pl.when(jj < NJJ - 1)
      def _():
        fire_gather(j + NBUF, b)

      fire_scatter(j, b)
    return 0
  lax.fori_loop(0, NJJ, outer, 0)
  for b in range(NBUF):
    wait_scatter((NJJ - 1) * NBUF + b, b)
  plsc.subcore_barrier()

  for k in range(ZB):
    r0 = base + k * C
    pltpu.sync_copy(acc_sh.at[pl.ds(r0, C)], zbuf)
    pltpu.sync_copy(zbuf, acc_out.at[c, pl.ds(r0, C)])

  def _drain_scalars(den_out, deg_out):
    for k in range(ZB):
      r0 = base + k * C
      pltpu.sync_copy(denom_sh.at[pl.ds(r0, C)], zflat)
      pltpu.sync_copy(zflat, den_out.at[pl.ds(r0, C)])
      pltpu.sync_copy(deg_sh.at[pl.ds(r0, C)], zflat)
      pltpu.sync_copy(zflat, deg_out.at[pl.ds(r0, C)])

  @pl.when(c == 0)
  def _():
    _drain_scalars(den0_out, deg0_out)

  @pl.when(c == 1)
  def _():
    _drain_scalars(den1_out, deg1_out)


def _tc_mid_body(acc0, acc1, den0, den1, deg0, deg1, h, ploop,
                 b1, gamma, beta, w2t, g_ref, dinv_ref):
  p = ploop[...]
  acc = acc0[...] + acc1[...] + p * h[...]
  den = den0[...] + den1[...] + p
  o = acc / (den + 1e-16) + b1[...]
  r = jnp.maximum(o, 0.0)
  mu = jnp.mean(r, axis=1, keepdims=True)
  var = jnp.mean((r - mu) ** 2, axis=1, keepdims=True)
  y = (r - mu) * lax.rsqrt(var + 1e-5) * gamma[...] + beta[...]
  h2 = jnp.sum(y * w2t[...], axis=1, keepdims=True)
  deg = deg0[...] + deg1[...] + 1.0
  dinv = lax.rsqrt(jnp.maximum(deg, 1.0))
  g_ref[...] = dinv * h2
  dinv_ref[...] = dinv


def _tc_mid(acc0, acc1, den0, den1, deg0, deg1, h, ploop, b1, gamma, beta, w2t):
  grid = (NP // BR,)
  row = lambda i: (i, 0)
  const = lambda i: (0, 0)
  return pl.pallas_call(
      _tc_mid_body,
      grid=grid,
      in_specs=[
          pl.BlockSpec((BR, H), row),
          pl.BlockSpec((BR, H), row),
          pl.BlockSpec((BR, 1), row),
          pl.BlockSpec((BR, 1), row),
          pl.BlockSpec((BR, 1), row),
          pl.BlockSpec((BR, 1), row),
          pl.BlockSpec((BR, H), row),
          pl.BlockSpec((BR, 1), row),
          pl.BlockSpec((1, H), const),
          pl.BlockSpec((1, H), const),
          pl.BlockSpec((1, H), const),
          pl.BlockSpec((1, H), const),
      ],
      out_specs=[
          pl.BlockSpec((BR, 1), row),
          pl.BlockSpec((BR, 1), row),
      ],
      out_shape=[
          jax.ShapeDtypeStruct((NP, 1), jnp.float32),
          jax.ShapeDtypeStruct((NP, 1), jnp.float32),
      ],
  )(acc0, acc1, den0, den1, deg0, deg1, h, ploop, b1, gamma, beta, w2t)


@functools.partial(
    pl.kernel,
    out_type=[
        jax.ShapeDtypeStruct((NP,), jnp.float32),
        jax.ShapeDtypeStruct((NP,), jnp.float32),
    ],
    mesh=_mesh,
    scratch_types=[
        pltpu.VMEM((NCH, C), jnp.int32),
        pltpu.VMEM((NCH, C), jnp.int32),
        pltpu.VMEM((NBUF, C), jnp.float32),
        pltpu.VMEM((NBUF, C), jnp.float32),
        pltpu.VMEM((C,), jnp.float32),
        pltpu.VMEM_SHARED((NP,), jnp.float32),
    ] + [pltpu.SemaphoreType.DMA] * (2 * NBUF),
    compiler_params=pltpu.CompilerParams(use_tc_tiling_on_sc=False),
)
def _sc_gcn(src_hbm, dst_hbm, g_hbm, acc2_0_out, acc2_1_out,
            sidx, didx, gval, sval, zflat, acc2_sh, *sems):
  gsem = sems[:NBUF]
  ssem = sems[NBUF:]
  c = lax.axis_index("c")
  s = lax.axis_index("s")
  wid = c * NS + s

  zero16 = jnp.zeros((L,), jnp.float32)
  for v in range(C // L):
    zflat[pl.ds(v * L, L)] = zero16
  base = s * RPT
  for k in range(ZB):
    pltpu.sync_copy(zflat, acc2_sh.at[pl.ds(base + k * C, C)])
  plsc.subcore_barrier()

  pltpu.sync_copy(src_hbm.at[wid], sidx)
  pltpu.sync_copy(dst_hbm.at[wid], didx)

  def fire_gather(j, b):
    pltpu.async_copy(g_hbm.at[sidx.at[j]], gval.at[b], gsem[b])

  def wait_gather(j, b):
    pltpu.make_async_copy(g_hbm.at[sidx.at[j]], gval.at[b], gsem[b]).wait()

  def fire_scatter(j, b):
    pltpu.async_copy(sval.at[b], acc2_sh.at[didx.at[j]], ssem[b], add=True)

  def wait_scatter(j, b):
    pltpu.make_async_copy(sval.at[b], acc2_sh.at[didx.at[j]], ssem[b]).wait()

  for b in range(NBUF):
    fire_gather(b, b)

  def outer(jj, _):
    for b in range(NBUF):
      j = jj * NBUF + b
      wait_gather(j, b)

      @pl.when(jj > 0)
      def _():
        wait_scatter(j - NBUF, b)

      def cp(v, _):
        sval[b, pl.ds(v * L, L)] = gval[b, pl.ds(v * L, L)]
        return 0
      lax.fori_loop(0, C // L, cp, 0)

      @pl.when(jj < NJJ - 1)
      def _():
        fire_gather(j + NBUF, b)

      fire_scatter(j, b)
    return 0
  lax.fori_loop(0, NJJ, outer, 0)
  for b in range(NBUF):
    wait_scatter((NJJ - 1) * NBUF + b, b)
  plsc.subcore_barrier()

  def _drain(out_ref):
    for k in range(ZB):
      r0 = base + k * C
      pltpu.sync_copy(acc2_sh.at[pl.ds(r0, C)], zflat)
      pltpu.sync_copy(zflat, out_ref.at[pl.ds(r0, C)])

  @pl.when(c == 0)
  def _():
    _drain(acc2_0_out)

  @pl.when(c == 1)
  def _():
    _drain(acc2_1_out)


def _tc_fin_body(a0, a1, g, dinv, b2, out_ref):
  out_ref[...] = dinv[...] * (a0[...] + a1[...] + g[...]) + b2[...]


def _tc_fin(a0, a1, g, dinv, b2):
  grid = (NP // BR,)
  row = lambda i: (i, 0)
  return pl.pallas_call(
      _tc_fin_body,
      grid=grid,
      in_specs=[
          pl.BlockSpec((BR, 1), row),
          pl.BlockSpec((BR, 1), row),
          pl.BlockSpec((BR, 1), row),
          pl.BlockSpec((BR, 1), row),
          pl.BlockSpec((1, 1), lambda i: (0, 0)),
      ],
      out_specs=pl.BlockSpec((BR, 1), row),
      out_shape=jax.ShapeDtypeStruct((NP, 1), jnp.float32),
  )(a0, a1, g, dinv, b2)


def kernel(x, edge_index, W1, a_src, a_dst, b1, gamma, beta, W2, b2):
  xp = jnp.pad(x, ((0, NP - N), (0, 0)))
  src3 = edge_index[0].reshape(NW, NCH, C)
  dst3 = edge_index[1].reshape(NW, NCH, C)

  h, as_, ad_, ploop = _tc_pre(xp, W1, a_src, a_dst)

  acc_p, den0, den1, deg0, deg1 = _sc_gat(
      src3, dst3, as_.reshape(NP), ad_.reshape(NP), h)

  g2, dinv2 = _tc_mid(
      acc_p[0], acc_p[1],
      den0.reshape(NP, 1), den1.reshape(NP, 1),
      deg0.reshape(NP, 1), deg1.reshape(NP, 1),
      h, ploop,
      b1.reshape(1, H), gamma.reshape(1, H), beta.reshape(1, H),
      W2.reshape(1, H))

  acc2_0, acc2_1 = _sc_gcn(src3, dst3, g2.reshape(NP))

  outp = _tc_fin(acc2_0.reshape(NP, 1), acc2_1.reshape(NP, 1),
                 g2, dinv2, b2.reshape(1, 1))
  return outp[:N, 0]
